# Initial kernel scaffold; baseline (speedup 1.0000x reference)
import jax
import jax.numpy as jnp
from jax.experimental import pallas as pl


def kernel(nodes, edges, Ws, bs, Wd, bd):
    raise NotImplementedError("write your pallas kernel here")


# speedup vs baseline: 14.5804x; 14.5804x over previous
import jax
import jax.numpy as jnp
from jax import lax
from jax.experimental import pallas as pl
from jax.experimental.pallas import tpu as pltpu
from jax.experimental.pallas import tpu_sc as plsc

N = 50000
C = 4
E = 160000
D = 128
P = 128
O = 128

NC = 2
NT = 16
LANES = 16

BN = 1024
NB = 49
N_PAD = NB * BN
ROWS_TILE = N_PAD // NT

CHUNK = 128
E_TILE = 10240
NCHUNK = E_TILE // CHUNK
E_PAD = NT * E_TILE

NR = 7
R_ROWS = N_PAD // NR
TILE_R = R_ROWS // NT
IB = 112
CBUF = E_TILE + CHUNK

_MESH = plsc.VectorSubcoreMesh(
    core_axis_name="c", subcore_axis_name="s", num_cores=NC, num_subcores=NT)


def _deg_body(edges_ref, deg_ref, ones_v, idx_v, bounce_v, acc0, acc1, sem):
    c = lax.axis_index("c")
    t = lax.axis_index("s")

    def fill(i, _):
        ones_v[pl.ds(i * LANES, LANES)] = jnp.ones((LANES,), jnp.float32)
        return 0

    lax.fori_loop(0, ROWS_TILE // LANES, fill, 0)

    sl = pl.ds(t * ROWS_TILE, ROWS_TILE)
    pltpu.sync_copy(ones_v, acc0.at[sl])
    pltpu.sync_copy(ones_v, acc1.at[sl])
    plsc.subcore_barrier()

    for kc, acc in ((0, acc0), (1, acc1)):
        k = 2 * c + kc
        pltpu.sync_copy(edges_ref.at[k, 1, t], idx_v)

        def upd(j, _):
            pltpu.sync_copy(ones_v.at[pl.ds(0, CHUNK)], acc.at[idx_v.at[j]],
                            add=True)
            return 0

        lax.fori_loop(0, NCHUNK, upd, 0)

    plsc.subcore_barrier()
    base = 2 * c * N_PAD + t * ROWS_TILE
    pltpu.sync_copy(acc0.at[sl], bounce_v)
    pltpu.sync_copy(bounce_v, deg_ref.at[pl.ds(base, ROWS_TILE)])
    pltpu.sync_copy(acc1.at[sl], bounce_v)
    pltpu.sync_copy(bounce_v, deg_ref.at[pl.ds(base + N_PAD, ROWS_TILE)])


def _deg_call(edges_pad):
    return pl.kernel(
        _deg_body,
        out_type=jax.ShapeDtypeStruct((C * N_PAD,), jnp.float32),
        mesh=_MESH,
        compiler_params=pltpu.CompilerParams(needs_layout_passes=False),
        scratch_types=[
            pltpu.VMEM((ROWS_TILE,), jnp.float32),
            pltpu.VMEM((NCHUNK, CHUNK), jnp.int32),
            pltpu.VMEM((ROWS_TILE,), jnp.float32),
            pltpu.VMEM_SHARED((N_PAD,), jnp.float32),
            pltpu.VMEM_SHARED((N_PAD,), jnp.float32),
            pltpu.SemaphoreType.DMA,
        ],
    )(edges_pad)


def _prescale_body(x_ref, w_ref, deg_ref, hs_ref, dinv_ref):
    h = jnp.dot(x_ref[...], w_ref[0], preferred_element_type=jnp.float32)
    dinv = lax.rsqrt(deg_ref[0, 0])
    dinv_ref[0, 0] = dinv
    hs_ref[0] = h * dinv[:, None]


def _prescale_call(nodes_pad, Ws, deg3):
    return pl.pallas_call(
        _prescale_body,
        grid=(NB, C),
        in_specs=[
            pl.BlockSpec((BN, D), lambda i, k: (i, 0)),
            pl.BlockSpec((1, D, P), lambda i, k: (k, 0, 0)),
            pl.BlockSpec((1, 1, BN), lambda i, k: (k, 0, i)),
        ],
        out_specs=[
            pl.BlockSpec((1, BN, P), lambda i, k: (k, i, 0)),
            pl.BlockSpec((1, 1, BN), lambda i, k: (k, 0, i)),
        ],
        out_shape=[
            jax.ShapeDtypeStruct((C, N_PAD, P), jnp.float32),
            jax.ShapeDtypeStruct((C, 1, N_PAD), jnp.float32),
        ],
    )(nodes_pad, Ws, deg3)


def _edge_body(edges_ref, hs_ref, a_ref, src_v, dst_v, srcbuf, offbuf,
               srcchunk_v, offchunk_v, rows_v, bounce_v, acc, sem):
    c = lax.axis_index("c")
    t = lax.axis_index("s")

    for kc in range(2):
        k = 2 * c + kc
        ebase = (k * 2 * NT + t) * E_TILE
        pltpu.sync_copy(edges_ref.at[pl.ds(ebase, E_TILE)], src_v)
        pltpu.sync_copy(edges_ref.at[pl.ds(ebase + NT * E_TILE, E_TILE)],
                        dst_v)
        hs_k = hs_ref.at[k]
        a_k = a_ref.at[k]
        for r in range(NR):
            lo = r * R_ROWS

            def init_step(i, _):
                gsl = pl.ds(lo + t * TILE_R + i * IB, IB)
                asl = pl.ds(t * TILE_R + i * IB, IB)
                pltpu.sync_copy(hs_k.at[gsl], bounce_v)
                pltpu.sync_copy(bounce_v, acc.at[asl])
                return 0

            lax.fori_loop(0, TILE_R // IB, init_step, 0)

            def filt(i, ptr):
                s = src_v[pl.ds(i * LANES, LANES)]
                d = dst_v[pl.ds(i * LANES, LANES)]
                m = (d >= lo) & (d < lo + R_ROWS)
                mi = m.astype(jnp.int32)
                pos = ptr + plsc.cumsum(mi) - mi
                plsc.store_scatter(srcbuf, [pos], s, mask=m)
                plsc.store_scatter(offbuf, [pos], d - lo, mask=m)
                return ptr + jnp.sum(mi)

            cnt = lax.fori_loop(0, E_TILE // LANES, filt, jnp.int32(0))

            sent_src = N + lax.iota(jnp.int32, LANES) * 8
            sent_off = lax.iota(jnp.int32, LANES)
            npad = (-cnt) % CHUNK
            for i in range(CHUNK // LANES):
                pos = cnt + i * LANES + lax.iota(jnp.int32, LANES)
                pm = pos < cnt + npad
                plsc.store_scatter(srcbuf, [pos], sent_src, mask=pm)
                plsc.store_scatter(offbuf, [pos], sent_off, mask=pm)
            nchunks = (cnt + npad) // CHUNK
            plsc.subcore_barrier()

            def copy16(i, dst_ref, src_ref, base_):
                dst_ref[pl.ds(i * LANES, LANES)] = (
                    src_ref[pl.ds(base_ + i * LANES, LANES)])

            def chunk_step(j, _):
                for i in range(CHUNK // LANES):
                    copy16(i, srcchunk_v, srcbuf, j * CHUNK)
                    copy16(i, offchunk_v, offbuf, j * CHUNK)
                pltpu.async_copy(hs_k.at[srcchunk_v], rows_v, sem).wait()
                pltpu.sync_copy(rows_v, acc.at[offchunk_v], add=True)
                return 0

            lax.fori_loop(0, nchunks, chunk_step, 0)
            plsc.subcore_barrier()

            def out_step(i, _):
                gsl = pl.ds(lo + t * TILE_R + i * IB, IB)
                asl = pl.ds(t * TILE_R + i * IB, IB)
                pltpu.sync_copy(acc.at[asl], bounce_v)
                pltpu.sync_copy(bounce_v, a_k.at[gsl])
                return 0

            lax.fori_loop(0, TILE_R // IB, out_step, 0)


def _edge_call(edges_flat, hs):
    return pl.kernel(
        _edge_body,
        out_type=jax.ShapeDtypeStruct((C, N_PAD, P), jnp.float32),
        mesh=_MESH,
        compiler_params=pltpu.CompilerParams(needs_layout_passes=False),
        scratch_types=[
            pltpu.VMEM((E_TILE,), jnp.int32),
            pltpu.VMEM((E_TILE,), jnp.int32),
            pltpu.VMEM((CBUF,), jnp.int32),
            pltpu.VMEM((CBUF,), jnp.int32),
            pltpu.VMEM((CHUNK,), jnp.int32),
            pltpu.VMEM((CHUNK,), jnp.int32),
            pltpu.VMEM((CHUNK, P), jnp.float32),
            pltpu.VMEM((IB, P), jnp.float32),
            pltpu.VMEM_SHARED((R_ROWS, P), jnp.float32),
            pltpu.SemaphoreType.DMA,
        ],
    )(edges_flat, hs)


def _epilogue_body(a_ref, dinv_ref, bs_ref, wd_ref, bd_ref, out_ref):
    total = None
    for k in range(C):
        y = jnp.maximum(
            a_ref[k] * dinv_ref[k, 0][:, None] + bs_ref[k][None, :], 0.0)
        d = jnp.dot(y, wd_ref[k], preferred_element_type=jnp.float32)
        total = d if total is None else total + d
    out_ref[...] = jnp.maximum(total + bd_ref[0][None, :], 0.0)


def _epilogue_call(A, dinv3, bs, Wd3, bd2):
    return pl.pallas_call(
        _epilogue_body,
        grid=(NB,),
        in_specs=[
            pl.BlockSpec((C, BN, P), lambda i: (0, i, 0)),
            pl.BlockSpec((C, 1, BN), lambda i: (0, 0, i)),
            pl.BlockSpec((C, P), lambda i: (0, 0)),
            pl.BlockSpec((C, P, O), lambda i: (0, 0, 0)),
            pl.BlockSpec((1, O), lambda i: (0, 0)),
        ],
        out_specs=pl.BlockSpec((BN, O), lambda i: (i, 0)),
        out_shape=jax.ShapeDtypeStruct((N, O), jnp.float32),
    )(A, dinv3, bs, Wd3, bd2)


def kernel(nodes, edges, Ws, bs, Wd, bd):
    pad_n = E_PAD - E
    sent = N + (jnp.arange(pad_n, dtype=jnp.int32) % 128)
    sent = jnp.broadcast_to(sent, (C, 2, pad_n))
    edges_pad = jnp.concatenate([edges, sent], axis=2)
    edges_5d = edges_pad.reshape(C, 2, NT, NCHUNK, CHUNK)
    edges_flat = edges_pad.reshape(-1)

    nodes_pad = jnp.pad(nodes, ((0, N_PAD - N), (0, 0)))

    deg = _deg_call(edges_5d)
    hs, dinv3 = _prescale_call(nodes_pad, Ws, deg.reshape(C, 1, N_PAD))
    A = _edge_call(edges_flat, hs)
    out = _epilogue_call(A, dinv3, bs, Wd.reshape(C, P, O),
                         bd.reshape(1, O))
    return out


# speedup vs baseline: 14.6082x; 1.0019x over previous
import jax
import jax.numpy as jnp
from jax import lax
from jax.experimental import pallas as pl
from jax.experimental.pallas import tpu as pltpu
from jax.experimental.pallas import tpu_sc as plsc

N = 50000
C = 4
E = 160000
D = 128
P = 128
O = 128

NC = 2
NT = 16
LANES = 16

BN = 1024
NB = 49
N_PAD = NB * BN
ROWS_TILE = N_PAD // NT

CHUNK = 128
E_TILE = 10240
NCHUNK = E_TILE // CHUNK
E_PAD = NT * E_TILE

NR = 7
R_ROWS = N_PAD // NR
TILE_R = R_ROWS // NT
IB = 56
SEG = 1280
CBUF = E_TILE + CHUNK

_MESH = plsc.VectorSubcoreMesh(
    core_axis_name="c", subcore_axis_name="s", num_cores=NC, num_subcores=NT)


def _deg_body(edges_ref, deg_ref, ones_v, idx_v, bounce_v, acc0, acc1, sem):
    c = lax.axis_index("c")
    t = lax.axis_index("s")

    def fill(i, _):
        ones_v[pl.ds(i * LANES, LANES)] = jnp.ones((LANES,), jnp.float32)
        return 0

    lax.fori_loop(0, ROWS_TILE // LANES, fill, 0)

    sl = pl.ds(t * ROWS_TILE, ROWS_TILE)
    pltpu.sync_copy(ones_v, acc0.at[sl])
    pltpu.sync_copy(ones_v, acc1.at[sl])
    plsc.subcore_barrier()

    for kc, acc in ((0, acc0), (1, acc1)):
        k = 2 * c + kc
        pltpu.sync_copy(edges_ref.at[k, 1, t], idx_v)

        def upd(j, _):
            pltpu.sync_copy(ones_v.at[pl.ds(0, CHUNK)], acc.at[idx_v.at[j]],
                            add=True)
            return 0

        lax.fori_loop(0, NCHUNK, upd, 0)

    plsc.subcore_barrier()
    base = 2 * c * N_PAD + t * ROWS_TILE
    pltpu.sync_copy(acc0.at[sl], bounce_v)
    pltpu.sync_copy(bounce_v, deg_ref.at[pl.ds(base, ROWS_TILE)])
    pltpu.sync_copy(acc1.at[sl], bounce_v)
    pltpu.sync_copy(bounce_v, deg_ref.at[pl.ds(base + N_PAD, ROWS_TILE)])


def _deg_call(edges_pad):
    return pl.kernel(
        _deg_body,
        out_type=jax.ShapeDtypeStruct((C * N_PAD,), jnp.float32),
        mesh=_MESH,
        compiler_params=pltpu.CompilerParams(needs_layout_passes=False),
        scratch_types=[
            pltpu.VMEM((ROWS_TILE,), jnp.float32),
            pltpu.VMEM((NCHUNK, CHUNK), jnp.int32),
            pltpu.VMEM((ROWS_TILE,), jnp.float32),
            pltpu.VMEM_SHARED((N_PAD,), jnp.float32),
            pltpu.VMEM_SHARED((N_PAD,), jnp.float32),
            pltpu.SemaphoreType.DMA,
        ],
    )(edges_pad)


def _prescale_body(x_ref, w_ref, deg_ref, hs_ref, dinv_ref):
    h = jnp.dot(x_ref[...], w_ref[0], preferred_element_type=jnp.float32)
    dinv = lax.rsqrt(deg_ref[0, 0])
    dinv_ref[0, 0] = dinv
    hs_ref[0] = h * dinv[:, None]


def _prescale_call(nodes_pad, Ws, deg3):
    return pl.pallas_call(
        _prescale_body,
        grid=(NB, C),
        in_specs=[
            pl.BlockSpec((BN, D), lambda i, k: (i, 0)),
            pl.BlockSpec((1, D, P), lambda i, k: (k, 0, 0)),
            pl.BlockSpec((1, 1, BN), lambda i, k: (k, 0, i)),
        ],
        out_specs=[
            pl.BlockSpec((1, BN, P), lambda i, k: (k, i, 0)),
            pl.BlockSpec((1, 1, BN), lambda i, k: (k, 0, i)),
        ],
        out_shape=[
            jax.ShapeDtypeStruct((C, N_PAD, P), jnp.float32),
            jax.ShapeDtypeStruct((C, 1, N_PAD), jnp.float32),
        ],
    )(nodes_pad, Ws, deg3)


def _edge_body(edges_ref, hs_ref, a_ref, sseg_v, dseg_v, srcbuf, offbuf,
               offchunk_v, rows2_v, bounce_v, acc, gsem, ssem):
    c = lax.axis_index("c")
    t = lax.axis_index("s")

    for kc in range(2):
        k = 2 * c + kc
        ebase = (k * 2 * NT + t) * E_TILE
        hs_k = hs_ref.at[k]
        a_k = a_ref.at[k]
        for r in range(NR):
            lo = r * R_ROWS

            def init_step(i, _):
                gsl = pl.ds(lo + t * TILE_R + i * IB, IB)
                asl = pl.ds(t * TILE_R + i * IB, IB)
                pltpu.sync_copy(hs_k.at[gsl], bounce_v)
                pltpu.sync_copy(bounce_v, acc.at[asl])
                return 0

            lax.fori_loop(0, TILE_R // IB, init_step, 0)

            def seg_step(s_i, ptr0):
                pltpu.sync_copy(
                    edges_ref.at[pl.ds(ebase + s_i * SEG, SEG)], sseg_v)
                pltpu.sync_copy(
                    edges_ref.at[pl.ds(ebase + NT * E_TILE + s_i * SEG, SEG)],
                    dseg_v)

                def filt(i, ptr):
                    s = sseg_v[pl.ds(i * LANES, LANES)]
                    d = dseg_v[pl.ds(i * LANES, LANES)]
                    m = (d >= lo) & (d < lo + R_ROWS)
                    mi = m.astype(jnp.int32)
                    pos = ptr + plsc.cumsum(mi) - mi
                    plsc.store_scatter(srcbuf, [pos], s, mask=m)
                    plsc.store_scatter(offbuf, [pos], d - lo, mask=m)
                    return ptr + jnp.sum(mi)

                return lax.fori_loop(0, SEG // LANES, filt, ptr0, unroll=4)

            cnt = lax.fori_loop(0, E_TILE // SEG, seg_step, jnp.int32(0))

            sent_src = N + lax.iota(jnp.int32, LANES) * 8
            sent_off = lax.iota(jnp.int32, LANES)
            npad = (-cnt) % CHUNK
            for i in range(CHUNK // LANES):
                pos = cnt + i * LANES + lax.iota(jnp.int32, LANES)
                pm = pos < cnt + npad
                plsc.store_scatter(srcbuf, [pos], sent_src, mask=pm)
                plsc.store_scatter(offbuf, [pos], sent_off, mask=pm)
            nchunks = (cnt + npad) // CHUNK
            plsc.subcore_barrier()

            def fire_gather(j):
                par = j % 2
                oc = offchunk_v.at[par]
                for i in range(CHUNK // LANES):
                    oc[pl.ds(i * LANES, LANES)] = (
                        offbuf[pl.ds(j * CHUNK + i * LANES, LANES)])
                pltpu.async_copy(
                    hs_k.at[srcbuf.at[pl.ds(j * CHUNK, CHUNK)]],
                    rows2_v.at[pl.ds(par * CHUNK, CHUNK)], gsem.at[par])

            def wait_gather(j):
                par = j % 2
                pltpu.make_async_copy(
                    hs_k.at[pl.ds(0, CHUNK)],
                    rows2_v.at[pl.ds(par * CHUNK, CHUNK)],
                    gsem.at[par]).wait()

            def fire_scatter(j):
                par = j % 2
                pltpu.async_copy(rows2_v.at[pl.ds(par * CHUNK, CHUNK)],
                                 acc.at[offchunk_v.at[par]], ssem.at[par],
                                 add=True)

            def wait_scatter(j):
                par = j % 2
                pltpu.make_async_copy(
                    hs_k.at[pl.ds(0, CHUNK)],
                    rows2_v.at[pl.ds(par * CHUNK, CHUNK)],
                    ssem.at[par]).wait()

            @pl.when(nchunks > 0)
            def _():
                fire_gather(0)

            def pipe(j, _):
                @pl.when(j >= 1)
                def _():
                    wait_scatter(j - 1)

                @pl.when(j + 1 < nchunks)
                def _():
                    fire_gather(j + 1)

                wait_gather(j)
                fire_scatter(j)
                return 0

            lax.fori_loop(0, nchunks, pipe, 0)

            @pl.when(nchunks > 0)
            def _():
                wait_scatter(nchunks - 1)

            plsc.subcore_barrier()

            def out_step(i, _):
                gsl = pl.ds(lo + t * TILE_R + i * IB, IB)
                asl = pl.ds(t * TILE_R + i * IB, IB)
                pltpu.sync_copy(acc.at[asl], bounce_v)
                pltpu.sync_copy(bounce_v, a_k.at[gsl])
                return 0

            lax.fori_loop(0, TILE_R // IB, out_step, 0)


def _edge_call(edges_flat, hs):
    return pl.kernel(
        _edge_body,
        out_type=jax.ShapeDtypeStruct((C, N_PAD, P), jnp.float32),
        mesh=_MESH,
        compiler_params=pltpu.CompilerParams(needs_layout_passes=False),
        scratch_types=[
            pltpu.VMEM((SEG,), jnp.int32),
            pltpu.VMEM((SEG,), jnp.int32),
            pltpu.VMEM((CBUF,), jnp.int32),
            pltpu.VMEM((CBUF,), jnp.int32),
            pltpu.VMEM((2, CHUNK), jnp.int32),
            pltpu.VMEM((2 * CHUNK, P), jnp.float32),
            pltpu.VMEM((IB, P), jnp.float32),
            pltpu.VMEM_SHARED((R_ROWS, P), jnp.float32),
            pltpu.SemaphoreType.DMA((2,)),
            pltpu.SemaphoreType.DMA((2,)),
        ],
    )(edges_flat, hs)


def _epilogue_body(a_ref, dinv_ref, bs_ref, wd_ref, bd_ref, out_ref):
    total = None
    for k in range(C):
        y = jnp.maximum(
            a_ref[k] * dinv_ref[k, 0][:, None] + bs_ref[k][None, :], 0.0)
        d = jnp.dot(y, wd_ref[k], preferred_element_type=jnp.float32)
        total = d if total is None else total + d
    out_ref[...] = jnp.maximum(total + bd_ref[0][None, :], 0.0)


def _epilogue_call(A, dinv3, bs, Wd3, bd2):
    return pl.pallas_call(
        _epilogue_body,
        grid=(NB,),
        in_specs=[
            pl.BlockSpec((C, BN, P), lambda i: (0, i, 0)),
            pl.BlockSpec((C, 1, BN), lambda i: (0, 0, i)),
            pl.BlockSpec((C, P), lambda i: (0, 0)),
            pl.BlockSpec((C, P, O), lambda i: (0, 0, 0)),
            pl.BlockSpec((1, O), lambda i: (0, 0)),
        ],
        out_specs=pl.BlockSpec((BN, O), lambda i: (i, 0)),
        out_shape=jax.ShapeDtypeStruct((N, O), jnp.float32),
    )(A, dinv3, bs, Wd3, bd2)


def kernel(nodes, edges, Ws, bs, Wd, bd):
    pad_n = E_PAD - E
    sent = N + (jnp.arange(pad_n, dtype=jnp.int32) % 128)
    sent = jnp.broadcast_to(sent, (C, 2, pad_n))
    edges_pad = jnp.concatenate([edges, sent], axis=2)
    edges_5d = edges_pad.reshape(C, 2, NT, NCHUNK, CHUNK)
    edges_flat = edges_pad.reshape(-1)

    nodes_pad = jnp.pad(nodes, ((0, N_PAD - N), (0, 0)))

    deg = _deg_call(edges_5d)
    hs, dinv3 = _prescale_call(nodes_pad, Ws, deg.reshape(C, 1, N_PAD))
    A = _edge_call(edges_flat, hs)
    out = _epilogue_call(A, dinv3, bs, Wd.reshape(C, P, O),
                         bd.reshape(1, O))
    return out
